# baseline (speedup 1.0000x reference)
import jax
import jax.numpy as jnp
from jax.experimental import pallas as pl

HID = 64
NUM_GRAPHS = 64


def _lin(x, W, b):
    return x @ W.T + b


def _sage(x_src, x_dst, ei, p):
    src = ei[0]
    dst = ei[1]
    n = x_dst.shape[0]
    msg = x_src[src]
    agg = jax.ops.segment_sum(msg, dst, num_segments=n)
    cnt = jax.ops.segment_sum(jnp.ones((msg.shape[0], 1), msg.dtype), dst, num_segments=n)
    mean = agg / jnp.maximum(cnt, 1.0)
    return mean @ p['Wl'].T + p['bl'] + x_dst @ p['Wr'].T


def _copy_k(x_ref, o_ref):
    o_ref[...] = x_ref[...]


def kernel(x_phys, x_log, x_gate, ei_intra, ei_inter, ei_mapped, ei_part, ei_dep, batch, params):
    xp = jax.nn.relu(_lin(x_phys, params['phys_enc']['W'], params['phys_enc']['b']))
    xl = jax.nn.relu(_lin(x_log, params['log_enc']['W'], params['log_enc']['b']))
    c1 = params['conv1']
    p1 = _sage(xp, xp, ei_intra, c1['intra']) + _sage(xp, xp, ei_inter, c1['inter']) + _sage(xl, xp, ei_mapped, c1['mapped'])
    p1 = jax.nn.relu(p1)
    c2 = params['conv2']
    p2 = _sage(p1, p1, ei_intra, c2['intra']) + _sage(p1, p1, ei_inter, c2['inter']) + _sage(xl, p1, ei_mapped, c2['mapped'])
    p2 = jax.nn.relu(p2)
    p2 = pl.pallas_call(
        _copy_k, out_shape=jax.ShapeDtypeStruct(p2.shape, p2.dtype))(p2)
    gsc = _lin(p2, params['pool_gate']['W'], params['pool_gate']['b'])
    m = jax.ops.segment_max(gsc, batch, num_segments=NUM_GRAPHS)
    e = jnp.exp(gsc - m[batch])
    denom = jax.ops.segment_sum(e, batch, num_segments=NUM_GRAPHS)
    att = e / denom[batch]
    emb = jax.ops.segment_sum(att * p2, batch, num_segments=NUM_GRAPHS)
    a = params['actor']
    logits = _lin(jax.nn.relu(_lin(emb, a['W1'], a['b1'])), a['W2'], a['b2'])
    c = params['critic']
    value = _lin(jax.nn.relu(_lin(emb, c['W1'], c['b1'])), c['W2'], c['b2'])
    return logits, value


# speedup vs baseline: 4.8427x; 4.8427x over previous
import jax
import jax.numpy as jnp
from jax import lax
from jax.experimental import pallas as pl
from jax.experimental.pallas import tpu as pltpu
from jax.experimental.pallas import tpu_sc as plsc

f32 = jnp.float32
i32 = jnp.int32

HID = 64
GRAPHS = 64
NPHYS = 50000
NLOG = 25000
ACC_ROWS = 51200
CNT_ROWS = 50176
NPHYS_PAD = 50176
NLOG_PAD = 25600


def _mesh():
    return plsc.VectorSubcoreMesh(
        core_axis_name="c", subcore_axis_name="s", num_cores=2, num_subcores=16)


_SC_PARAMS = pltpu.CompilerParams(use_tc_tiling_on_sc=False)


def _counts_call(dsti, dste, dstm):
    n_i = dsti.shape[0] // (32 * 8)
    n_e = dste.shape[0] // (32 * 8)
    n_m = dstm.shape[0] // (32 * 8)

    def body(di_hbm, de_hbm, dm_hbm, out, didx, ones_v, zb, acc_i, acc_e, acc_m):
        cc = lax.axis_index("c")
        ss = lax.axis_index("s")
        wid = cc * 16 + ss
        for t in range(8):
            ones_v[pl.ds(t * 16, 16)] = jnp.ones((16,), f32)

        def zr(t, _):
            zb[pl.ds(t * 16, 16)] = jnp.zeros((16,), f32)
            return 0

        lax.fori_loop(0, 3136 // 16, zr, 0)
        for acc in (acc_i, acc_e, acc_m):
            pltpu.sync_copy(zb, acc.at[pl.ds(ss * 3136, 3136)])
        plsc.subcore_barrier()

        def do(dst_hbm, acc, n_sb, rows_per_w):
            def sb(j, _):
                pltpu.sync_copy(dst_hbm.at[pl.ds(wid * rows_per_w + j * 8, 8), :],
                                didx)
                for q in range(8):
                    pltpu.sync_copy(ones_v, acc.at[didx.at[q]], add=True)
                return 0

            lax.fori_loop(0, n_sb, sb, 0)

        do(di_hbm, acc_i, n_i, dsti.shape[0] // 32)
        do(de_hbm, acc_e, n_e, dste.shape[0] // 32)
        do(dm_hbm, acc_m, n_m, dstm.shape[0] // 32)
        plsc.subcore_barrier()
        for r, acc in enumerate((acc_i, acc_e, acc_m)):
            pltpu.sync_copy(acc.at[pl.ds(ss * 3136, 3136)],
                            out.at[cc, r, pl.ds(ss * 3136, 3136)])

    fn = pl.kernel(
        body,
        out_type=jax.ShapeDtypeStruct((2, 3, CNT_ROWS), f32),
        mesh=_mesh(),
        compiler_params=_SC_PARAMS,
        scratch_types=[pltpu.VMEM((8, 128), i32), pltpu.VMEM((128,), f32),
                       pltpu.VMEM((3136,), f32),
                       pltpu.VMEM_SHARED((CNT_ROWS,), f32),
                       pltpu.VMEM_SHARED((CNT_ROWS,), f32),
                       pltpu.VMEM_SHARED((CNT_ROWS,), f32)],
    )
    return fn(dsti, dste, dstm)


def _segsum_call(x2, srcr, dstr):
    rows_per_tile = srcr.shape[0] // 16
    n_sb = rows_per_tile // 8

    def body(x_hbm, s_hbm, d_hbm, out, sidx, didx, rows, zb, acc, sem):
        cc = lax.axis_index("c")
        ss = lax.axis_index("s")

        def zr(r, _):
            for c2 in range(2):
                zb[r, pl.ds(c2 * 16, 16)] = jnp.zeros((16,), f32)
            return 0

        lax.fori_loop(0, 128, zr, 0)
        for k in range(25):
            pltpu.sync_copy(zb, acc.at[pl.ds(ss * 3200 + k * 128, 128), :])
        plsc.subcore_barrier()

        def sb(j, _):
            row0 = ss * rows_per_tile + j * 8
            pltpu.sync_copy(s_hbm.at[pl.ds(row0, 8), :], sidx)
            pltpu.sync_copy(d_hbm.at[pl.ds(row0, 8), :], didx)
            for q in range(8):
                for g in range(8):
                    v = sidx[q, pl.ds(g * 16, 16)]
                    sidx[q, pl.ds(g * 16, 16)] = 2 * v + cc
            for q in range(8):
                pltpu.async_copy(x_hbm.at[sidx.at[q]], rows, sem).wait()
                pltpu.sync_copy(rows, acc.at[didx.at[q]], add=True)
            return 0

        lax.fori_loop(0, n_sb, sb, 0)
        plsc.subcore_barrier()
        pltpu.sync_copy(acc.at[pl.ds(ss * 3125, 3125), :],
                        out.at[cc, pl.ds(ss * 3125, 3125), :])

    fn = pl.kernel(
        body,
        out_type=jax.ShapeDtypeStruct((2, NPHYS, 32), f32),
        mesh=_mesh(),
        compiler_params=_SC_PARAMS,
        scratch_types=[pltpu.VMEM((8, 128), i32), pltpu.VMEM((8, 128), i32),
                       pltpu.VMEM((128, 32), f32), pltpu.VMEM((128, 32), f32),
                       pltpu.VMEM_SHARED((ACC_ROWS, 32), f32),
                       pltpu.SemaphoreType.DMA],
    )
    return fn(x2, srcr, dstr)


def _tc_encode(xpad, w8t, b):
    R = xpad.shape[0]
    BLK = 1024

    def body(x_ref, w_ref, b_ref, o_ref):
        o_ref[...] = jnp.maximum(
            jnp.dot(x_ref[...], w_ref[...], preferred_element_type=f32)
            + b_ref[...], 0.0)

    return pl.pallas_call(
        body, grid=(R // BLK,),
        in_specs=[pl.BlockSpec((BLK, 8), lambda i: (i, 0)),
                  pl.BlockSpec((8, HID), lambda i: (0, 0)),
                  pl.BlockSpec((1, HID), lambda i: (0, 0))],
        out_specs=pl.BlockSpec((BLK, HID), lambda i: (i, 0)),
        out_shape=jax.ShapeDtypeStruct((R, HID), f32))(xpad, w8t, b)


def _tc_recip(cnts):
    def body(c_ref, o_ref):
        cnt = c_ref[0] + c_ref[1]
        r = 1.0 / jnp.maximum(cnt, 1.0)
        r8 = jnp.concatenate([r, jnp.zeros((5, CNT_ROWS), f32)], axis=0)
        o_ref[...] = r8.T

    return pl.pallas_call(
        body,
        out_shape=jax.ShapeDtypeStruct((CNT_ROWS, 8), f32))(cnts)


def _tc_combine(silo, sihi, selo, sehi, smlo, smhi, recip, xin,
                wit, wet, wmt, wrt, bsum):
    BLK = 1000

    def body(a_r, b_r, c_r, d_r, e_r, f_r, rc_r, x_r, wi_r, we_r, wm_r, wr_r,
             bb_r, o_ref):
        rc = rc_r[...]
        si = jnp.concatenate([a_r[...], b_r[...]], axis=1)
        se = jnp.concatenate([c_r[...], d_r[...]], axis=1)
        sm = jnp.concatenate([e_r[...], f_r[...]], axis=1)
        acc = jnp.dot(si * rc[:, 0:1], wi_r[...], preferred_element_type=f32)
        acc += jnp.dot(se * rc[:, 1:2], we_r[...], preferred_element_type=f32)
        acc += jnp.dot(sm * rc[:, 2:3], wm_r[...], preferred_element_type=f32)
        acc += jnp.dot(x_r[...], wr_r[...], preferred_element_type=f32)
        o_ref[...] = jnp.maximum(acc + bb_r[...], 0.0)

    hspec = pl.BlockSpec((BLK, 32), lambda i: (i, 0))
    wspec = pl.BlockSpec((HID, HID), lambda i: (0, 0))
    return pl.pallas_call(
        body, grid=(NPHYS // BLK,),
        in_specs=[hspec, hspec, hspec, hspec, hspec, hspec,
                  pl.BlockSpec((BLK, 8), lambda i: (i, 0)),
                  pl.BlockSpec((BLK, HID), lambda i: (i, 0)),
                  wspec, wspec, wspec, wspec,
                  pl.BlockSpec((1, HID), lambda i: (0, 0))],
        out_specs=pl.BlockSpec((BLK, HID), lambda i: (i, 0)),
        out_shape=jax.ShapeDtypeStruct((NPHYS, HID), f32))(
            silo, sihi, selo, sehi, smlo, smhi, recip, xin,
            wit, wet, wmt, wrt, bsum)


def _tc_pool_a(p2, wpt, bp, batch_col):
    BLK = 1000

    def body(p_ref, w_ref, b_ref, bat_ref, g_ref, m_ref):
        i = pl.program_id(0)
        g = jnp.dot(p_ref[...], w_ref[...], preferred_element_type=f32) + b_ref[...]
        g_ref[...] = g
        gid = lax.broadcasted_iota(i32, (BLK, GRAPHS), 1)
        mask = bat_ref[...] == gid
        mc = jnp.max(jnp.where(mask, g, -1e30), axis=0, keepdims=True)

        @pl.when(i == 0)
        def _():
            m_ref[...] = mc

        @pl.when(i != 0)
        def _():
            m_ref[...] = jnp.maximum(m_ref[...], mc)

    return pl.pallas_call(
        body, grid=(NPHYS // BLK,),
        in_specs=[pl.BlockSpec((BLK, HID), lambda i: (i, 0)),
                  pl.BlockSpec((HID, 1), lambda i: (0, 0)),
                  pl.BlockSpec((1, 1), lambda i: (0, 0)),
                  pl.BlockSpec((BLK, 1), lambda i: (i, 0))],
        out_specs=[pl.BlockSpec((BLK, 1), lambda i: (i, 0)),
                   pl.BlockSpec((1, GRAPHS), lambda i: (0, 0))],
        out_shape=[jax.ShapeDtypeStruct((NPHYS, 1), f32),
                   jax.ShapeDtypeStruct((1, GRAPHS), f32)])(
            p2, wpt, bp, batch_col)


def _tc_pool_b(gsc, p2, batch_col, m):
    BLK = 1000

    def body(g_ref, p_ref, bat_ref, m_ref, den_ref, emb_ref):
        i = pl.program_id(0)
        gid = lax.broadcasted_iota(i32, (BLK, GRAPHS), 1)
        mask = bat_ref[...] == gid
        mf = mask.astype(f32)
        mrow = jnp.dot(mf, m_ref[...].T, preferred_element_type=f32)
        e = jnp.exp(g_ref[...] - mrow)
        we = jnp.where(mask, e, 0.0)
        dc = jnp.sum(we, axis=0, keepdims=True)
        ec = lax.dot_general(we, p_ref[...], (((0,), (0,)), ((), ())),
                             preferred_element_type=f32)

        @pl.when(i == 0)
        def _():
            den_ref[...] = dc
            emb_ref[...] = ec

        @pl.when(i != 0)
        def _():
            den_ref[...] = den_ref[...] + dc
            emb_ref[...] = emb_ref[...] + ec

    return pl.pallas_call(
        body, grid=(NPHYS // BLK,),
        in_specs=[pl.BlockSpec((BLK, 1), lambda i: (i, 0)),
                  pl.BlockSpec((BLK, HID), lambda i: (i, 0)),
                  pl.BlockSpec((BLK, 1), lambda i: (i, 0)),
                  pl.BlockSpec((1, GRAPHS), lambda i: (0, 0))],
        out_specs=[pl.BlockSpec((1, GRAPHS), lambda i: (0, 0)),
                   pl.BlockSpec((GRAPHS, HID), lambda i: (0, 0))],
        out_shape=[jax.ShapeDtypeStruct((1, GRAPHS), f32),
                   jax.ShapeDtypeStruct((GRAPHS, HID), f32)])(
            gsc, p2, batch_col, m)


def _tc_pool_c(emb, den, aw1t, ab1, aw2t, ab2, cw1t, cb1, cw2t, cb2):
    ACT = aw2t.shape[1]

    def body(e_ref, d_ref, a1, a1b, a2, a2b, c1, c1b, c2, c2b, lo_ref, v_ref):
        den_col = jnp.maximum(d_ref[...].T, 1e-20)
        e = e_ref[...] / den_col
        ha = jnp.maximum(jnp.dot(e, a1[...], preferred_element_type=f32) + a1b[...], 0.0)
        lo_ref[...] = jnp.dot(ha, a2[...], preferred_element_type=f32) + a2b[...]
        hc = jnp.maximum(jnp.dot(e, c1[...], preferred_element_type=f32) + c1b[...], 0.0)
        v_ref[...] = jnp.dot(hc, c2[...], preferred_element_type=f32) + c2b[...]

    return pl.pallas_call(
        body,
        out_shape=[jax.ShapeDtypeStruct((GRAPHS, ACT), f32),
                   jax.ShapeDtypeStruct((GRAPHS, 1), f32)])(
            emb, den, aw1t, ab1, aw2t, ab2, cw1t, cb1, cw2t, cb2)


def _prep_edges(ei, epad, nsrc):
    e = ei.shape[1]
    src = ei[0].astype(i32)
    dst = ei[1].astype(i32)
    pad = epad - e
    ar = jnp.arange(pad, dtype=i32)
    srcp = jnp.concatenate([src, ar % nsrc])
    dstp = jnp.concatenate([dst, NPHYS + (ar % 64)])
    return srcp.reshape(-1, 128), dstp.reshape(-1, 128)


def kernel(x_phys, x_log, x_gate, ei_intra, ei_inter, ei_mapped, ei_part,
           ei_dep, batch, params):
    del x_gate, ei_part, ei_dep

    si, di = _prep_edges(ei_intra, 819200, NPHYS)
    se, de = _prep_edges(ei_inter, 229376, NPHYS)
    sm, dm = _prep_edges(ei_mapped, 32768, NLOG)

    cnts = _counts_call(di, de, dm)
    recip = _tc_recip(cnts)

    xp_pad = jnp.pad(x_phys, ((0, NPHYS_PAD - NPHYS), (0, 3)))
    xl_pad = jnp.pad(x_log, ((0, NLOG_PAD - NLOG), (0, 7)))
    pe, le = params['phys_enc'], params['log_enc']
    wp8 = jnp.pad(pe['W'].T, ((0, 3), (0, 0)))
    wl8 = jnp.pad(le['W'].T, ((0, 7), (0, 0)))
    xp = _tc_encode(xp_pad, wp8, pe['b'].reshape(1, HID))
    xl = _tc_encode(xl_pad, wl8, le['b'].reshape(1, HID))
    xp2 = xp.reshape(2 * NPHYS_PAD, 32)
    xl2 = xl.reshape(2 * NLOG_PAD, 32)

    s1i = _segsum_call(xp2, si, di)
    s1e = _segsum_call(xp2, se, de)
    s1m = _segsum_call(xl2, sm, dm)
    c1 = params['conv1']
    wr1 = (c1['intra']['Wr'] + c1['inter']['Wr'] + c1['mapped']['Wr']).T
    b1 = (c1['intra']['bl'] + c1['inter']['bl'] + c1['mapped']['bl']).reshape(1, HID)
    p1 = _tc_combine(s1i[0], s1i[1], s1e[0], s1e[1], s1m[0], s1m[1], recip, xp,
                     c1['intra']['Wl'].T, c1['inter']['Wl'].T,
                     c1['mapped']['Wl'].T, wr1, b1)

    p12 = p1.reshape(2 * NPHYS, 32)
    s2i = _segsum_call(p12, si, di)
    s2e = _segsum_call(p12, se, de)
    c2 = params['conv2']
    wr2 = (c2['intra']['Wr'] + c2['inter']['Wr'] + c2['mapped']['Wr']).T
    b2 = (c2['intra']['bl'] + c2['inter']['bl'] + c2['mapped']['bl']).reshape(1, HID)
    p2 = _tc_combine(s2i[0], s2i[1], s2e[0], s2e[1], s1m[0], s1m[1], recip, p1,
                     c2['intra']['Wl'].T, c2['inter']['Wl'].T,
                     c2['mapped']['Wl'].T, wr2, b2)

    batch_col = batch.astype(i32).reshape(NPHYS, 1)
    pg = params['pool_gate']
    gsc, m = _tc_pool_a(p2, pg['W'].T, pg['b'].reshape(1, 1), batch_col)
    den, emb = _tc_pool_b(gsc, p2, batch_col, m)
    a, c = params['actor'], params['critic']
    logits, value = _tc_pool_c(
        emb, den, a['W1'].T, a['b1'].reshape(1, HID), a['W2'].T,
        a['b2'].reshape(1, -1), c['W1'].T, c['b1'].reshape(1, HID),
        c['W2'].T, c['b2'].reshape(1, 1))
    return logits, value


# speedup vs baseline: 7.9960x; 1.6511x over previous
import jax
import jax.numpy as jnp
from jax import lax
from jax.experimental import pallas as pl
from jax.experimental.pallas import tpu as pltpu
from jax.experimental.pallas import tpu_sc as plsc

f32 = jnp.float32
i32 = jnp.int32

HID = 64
GRAPHS = 64
NPHYS = 50000
NLOG = 25000
CNT_ROWS = 50176
NPHYS_PAD = 50176
NLOG_PAD = 25600


def _mesh():
    return plsc.VectorSubcoreMesh(
        core_axis_name="c", subcore_axis_name="s", num_cores=2, num_subcores=16)


_SC_PARAMS = pltpu.CompilerParams(use_tc_tiling_on_sc=False)


def _counts_call(dsti, dste, dstm, e_i, e_e, e_m):
    n_i = e_i // (32 * 1024)
    n_e = e_e // (32 * 1024)
    n_m = e_m // (32 * 1024)
    rw_i, rw_e, rw_m = e_i // (32 * 128), e_e // (32 * 128), e_m // (32 * 128)

    def body(di_hbm, de_hbm, dm_hbm, out, didx, ones_v, zb, acc_i, acc_e, acc_m):
        cc = lax.axis_index("c")
        ss = lax.axis_index("s")
        wid = cc * 16 + ss
        for t in range(8):
            ones_v[pl.ds(t * 16, 16)] = jnp.ones((16,), f32)

        def zr(t, _):
            zb[pl.ds(t * 16, 16)] = jnp.zeros((16,), f32)
            return 0

        lax.fori_loop(0, 3136 // 16, zr, 0)
        for acc in (acc_i, acc_e, acc_m):
            pltpu.sync_copy(zb, acc.at[pl.ds(ss * 3136, 3136)])
        plsc.subcore_barrier()

        def do(dst_hbm, acc, n_sb, rows_per_w):
            def sb(j, _):
                pltpu.sync_copy(dst_hbm.at[pl.ds(wid * rows_per_w + j * 8, 8), :],
                                didx)
                for q in range(8):
                    pltpu.sync_copy(ones_v, acc.at[didx.at[q]], add=True)
                return 0

            lax.fori_loop(0, n_sb, sb, 0)

        do(di_hbm, acc_i, n_i, rw_i)
        do(de_hbm, acc_e, n_e, rw_e)
        do(dm_hbm, acc_m, n_m, rw_m)
        plsc.subcore_barrier()
        for r, acc in enumerate((acc_i, acc_e, acc_m)):
            pltpu.sync_copy(acc.at[pl.ds(ss * 3136, 3136)],
                            out.at[cc, r, pl.ds(ss * 3136, 3136)])

    fn = pl.kernel(
        body,
        out_type=jax.ShapeDtypeStruct((2, 3, CNT_ROWS), f32),
        mesh=_mesh(),
        compiler_params=_SC_PARAMS,
        scratch_types=[pltpu.VMEM((8, 128), i32), pltpu.VMEM((128,), f32),
                       pltpu.VMEM((3136,), f32),
                       pltpu.VMEM_SHARED((CNT_ROWS,), f32),
                       pltpu.VMEM_SHARED((CNT_ROWS,), f32),
                       pltpu.VMEM_SHARED((CNT_ROWS,), f32)],
    )
    return fn(dsti, dste, dstm)


def _segsum_call(x2, src1d, dstr, e_pad):
    ept = e_pad // 16
    n_sb = ept // 1024
    n_sb2 = n_sb // 2

    def body(x_hbm, s_hbm, d_hbm, out, sidx, didx, rows, acc,
             semg, sems, semi, semz):
        cc = lax.axis_index("c")
        ss = lax.axis_index("s")

        def zr(r, _):
            for b in range(3):
                for c2 in range(2):
                    rows[b, r, pl.ds(c2 * 16, 16)] = jnp.zeros((16,), f32)
            return 0

        lax.fori_loop(0, 256, zr, 0)
        zd = []
        for k in range(12):
            zd.append(pltpu.async_copy(
                rows.at[k % 3], acc.at[pl.ds(ss * 3136 + k * 256, 256), :],
                semz))
        zd.append(pltpu.async_copy(
            rows.at[0, pl.ds(0, 64), :],
            acc.at[pl.ds(ss * 3136 + 3072, 64), :], semz))
        for d in zd:
            d.wait()
        plsc.subcore_barrier()

        base = ss * ept

        pltpu.async_copy(s_hbm.at[pl.ds(base, 1024)], sidx.at[0], semi)
        pltpu.async_copy(d_hbm.at[pl.ds(base // 128, 8), :], didx.at[0], semi)
        pltpu.async_copy(s_hbm.at[pl.ds(base + 1024, 1024)], sidx.at[1], semi)
        pltpu.async_copy(d_hbm.at[pl.ds(base // 128 + 8, 8), :], didx.at[1],
                         semi)

        def sb2(jj, _):
            for half in range(2):
                j = jj * 2 + half
                pltpu.make_async_copy(
                    s_hbm.at[pl.ds(base, 1024)], sidx.at[half], semi).wait()
                pltpu.make_async_copy(
                    d_hbm.at[pl.ds(base // 128, 8), :], didx.at[half],
                    semi).wait()
                for g in range(64):
                    v = sidx[half, pl.ds(g * 16, 16)]
                    sidx[half, pl.ds(g * 16, 16)] = 2 * v + cc

                def G(k, b):
                    return pltpu.async_copy(
                        x_hbm.at[sidx.at[half, pl.ds(k * 256, 256)]],
                        rows.at[b], semg)

                def SC(q, b):
                    return pltpu.async_copy(
                        rows.at[b, pl.ds((q % 2) * 128, 128), :],
                        acc.at[didx.at[half, q]], sems, add=True)

                gd = {k: G(k, k) for k in range(3)}
                sd = {}
                gd[0].wait()
                sd[0] = SC(0, 0)
                sd[1] = SC(1, 0)
                gd[1].wait()
                sd[2] = SC(2, 1)
                sd[3] = SC(3, 1)
                sd[0].wait()
                sd[1].wait()
                gd[3] = G(3, 0)
                gd[2].wait()
                sd[4] = SC(4, 2)
                sd[5] = SC(5, 2)
                gd[3].wait()
                sd[6] = SC(6, 0)
                sd[7] = SC(7, 0)
                nxt = base + (j + 2) * 1024

                @pl.when(j + 2 < n_sb)
                def _():
                    pltpu.async_copy(s_hbm.at[pl.ds(nxt, 1024)],
                                     sidx.at[half], semi)
                    pltpu.async_copy(d_hbm.at[pl.ds(nxt // 128, 8), :],
                                     didx.at[half], semi)
                for q in range(2, 8):
                    sd[q].wait()
            return 0

        lax.fori_loop(0, n_sb2, sb2, 0)
        plsc.subcore_barrier()
        pltpu.sync_copy(acc.at[pl.ds(ss * 3125, 3125), :],
                        out.at[cc, pl.ds(ss * 3125, 3125), :])

    fn = pl.kernel(
        body,
        out_type=jax.ShapeDtypeStruct((2, NPHYS, 32), f32),
        mesh=_mesh(),
        compiler_params=_SC_PARAMS,
        scratch_types=[pltpu.VMEM((2, 1024), i32), pltpu.VMEM((2, 8, 128), i32),
                       pltpu.VMEM((3, 256, 32), f32),
                       pltpu.VMEM_SHARED((CNT_ROWS, 32), f32),
                       pltpu.SemaphoreType.DMA, pltpu.SemaphoreType.DMA,
                       pltpu.SemaphoreType.DMA, pltpu.SemaphoreType.DMA],
    )
    return fn(x2, src1d, dstr)


def _tc_encode(xpad, w8t, b):
    R = xpad.shape[0]
    BLK = 1024

    def body(x_ref, w_ref, b_ref, o_ref):
        o_ref[...] = jnp.maximum(
            jnp.dot(x_ref[...], w_ref[...], preferred_element_type=f32)
            + b_ref[...], 0.0)

    return pl.pallas_call(
        body, grid=(R // BLK,),
        in_specs=[pl.BlockSpec((BLK, 8), lambda i: (i, 0)),
                  pl.BlockSpec((8, HID), lambda i: (0, 0)),
                  pl.BlockSpec((1, HID), lambda i: (0, 0))],
        out_specs=pl.BlockSpec((BLK, HID), lambda i: (i, 0)),
        out_shape=jax.ShapeDtypeStruct((R, HID), f32))(xpad, w8t, b)


def _tc_recip(cnts):
    def body(c_ref, o_ref):
        cnt = c_ref[0] + c_ref[1]
        r = 1.0 / jnp.maximum(cnt, 1.0)
        r8 = jnp.concatenate([r, jnp.zeros((5, CNT_ROWS), f32)], axis=0)
        o_ref[...] = r8.T

    return pl.pallas_call(
        body,
        out_shape=jax.ShapeDtypeStruct((CNT_ROWS, 8), f32))(cnts)


def _tc_combine(silo, sihi, selo, sehi, smlo, smhi, recip, xin,
                wit, wet, wmt, wrt, bsum):
    BLK = 1000

    def body(a_r, b_r, c_r, d_r, e_r, f_r, rc_r, x_r, wi_r, we_r, wm_r, wr_r,
             bb_r, o_ref):
        rc = rc_r[...]
        si = jnp.concatenate([a_r[...], b_r[...]], axis=1)
        se = jnp.concatenate([c_r[...], d_r[...]], axis=1)
        sm = jnp.concatenate([e_r[...], f_r[...]], axis=1)
        acc = jnp.dot(si * rc[:, 0:1], wi_r[...], preferred_element_type=f32)
        acc += jnp.dot(se * rc[:, 1:2], we_r[...], preferred_element_type=f32)
        acc += jnp.dot(sm * rc[:, 2:3], wm_r[...], preferred_element_type=f32)
        acc += jnp.dot(x_r[...], wr_r[...], preferred_element_type=f32)
        o_ref[...] = jnp.maximum(acc + bb_r[...], 0.0)

    hspec = pl.BlockSpec((BLK, 32), lambda i: (i, 0))
    wspec = pl.BlockSpec((HID, HID), lambda i: (0, 0))
    return pl.pallas_call(
        body, grid=(NPHYS // BLK,),
        in_specs=[hspec, hspec, hspec, hspec, hspec, hspec,
                  pl.BlockSpec((BLK, 8), lambda i: (i, 0)),
                  pl.BlockSpec((BLK, HID), lambda i: (i, 0)),
                  wspec, wspec, wspec, wspec,
                  pl.BlockSpec((1, HID), lambda i: (0, 0))],
        out_specs=pl.BlockSpec((BLK, HID), lambda i: (i, 0)),
        out_shape=jax.ShapeDtypeStruct((NPHYS, HID), f32))(
            silo, sihi, selo, sehi, smlo, smhi, recip, xin,
            wit, wet, wmt, wrt, bsum)


def _tc_pool_a(p2, wpt, bp, batch_col):
    BLK = 1000

    def body(p_ref, w_ref, b_ref, bat_ref, g_ref, m_ref):
        i = pl.program_id(0)
        g = jnp.dot(p_ref[...], w_ref[...], preferred_element_type=f32) + b_ref[...]
        g_ref[...] = g
        gid = lax.broadcasted_iota(i32, (BLK, GRAPHS), 1)
        mask = bat_ref[...] == gid
        mc = jnp.max(jnp.where(mask, g, -1e30), axis=0, keepdims=True)

        @pl.when(i == 0)
        def _():
            m_ref[...] = mc

        @pl.when(i != 0)
        def _():
            m_ref[...] = jnp.maximum(m_ref[...], mc)

    return pl.pallas_call(
        body, grid=(NPHYS // BLK,),
        in_specs=[pl.BlockSpec((BLK, HID), lambda i: (i, 0)),
                  pl.BlockSpec((HID, 1), lambda i: (0, 0)),
                  pl.BlockSpec((1, 1), lambda i: (0, 0)),
                  pl.BlockSpec((BLK, 1), lambda i: (i, 0))],
        out_specs=[pl.BlockSpec((BLK, 1), lambda i: (i, 0)),
                   pl.BlockSpec((1, GRAPHS), lambda i: (0, 0))],
        out_shape=[jax.ShapeDtypeStruct((NPHYS, 1), f32),
                   jax.ShapeDtypeStruct((1, GRAPHS), f32)])(
            p2, wpt, bp, batch_col)


def _tc_pool_b(gsc, p2, batch_col, m):
    BLK = 1000

    def body(g_ref, p_ref, bat_ref, m_ref, den_ref, emb_ref):
        i = pl.program_id(0)
        gid = lax.broadcasted_iota(i32, (BLK, GRAPHS), 1)
        mask = bat_ref[...] == gid
        mf = mask.astype(f32)
        mrow = jnp.dot(mf, m_ref[...].T, preferred_element_type=f32)
        e = jnp.exp(g_ref[...] - mrow)
        we = jnp.where(mask, e, 0.0)
        dc = jnp.sum(we, axis=0, keepdims=True)
        ec = lax.dot_general(we, p_ref[...], (((0,), (0,)), ((), ())),
                             preferred_element_type=f32)

        @pl.when(i == 0)
        def _():
            den_ref[...] = dc
            emb_ref[...] = ec

        @pl.when(i != 0)
        def _():
            den_ref[...] = den_ref[...] + dc
            emb_ref[...] = emb_ref[...] + ec

    return pl.pallas_call(
        body, grid=(NPHYS // BLK,),
        in_specs=[pl.BlockSpec((BLK, 1), lambda i: (i, 0)),
                  pl.BlockSpec((BLK, HID), lambda i: (i, 0)),
                  pl.BlockSpec((BLK, 1), lambda i: (i, 0)),
                  pl.BlockSpec((1, GRAPHS), lambda i: (0, 0))],
        out_specs=[pl.BlockSpec((1, GRAPHS), lambda i: (0, 0)),
                   pl.BlockSpec((GRAPHS, HID), lambda i: (0, 0))],
        out_shape=[jax.ShapeDtypeStruct((1, GRAPHS), f32),
                   jax.ShapeDtypeStruct((GRAPHS, HID), f32)])(
            gsc, p2, batch_col, m)


def _tc_pool_c(emb, den, aw1t, ab1, aw2t, ab2, cw1t, cb1, cw2t, cb2):
    ACT = aw2t.shape[1]

    def body(e_ref, d_ref, a1, a1b, a2, a2b, c1, c1b, c2, c2b, lo_ref, v_ref):
        den_col = jnp.maximum(d_ref[...].T, 1e-20)
        e = e_ref[...] / den_col
        ha = jnp.maximum(jnp.dot(e, a1[...], preferred_element_type=f32) + a1b[...], 0.0)
        lo_ref[...] = jnp.dot(ha, a2[...], preferred_element_type=f32) + a2b[...]
        hc = jnp.maximum(jnp.dot(e, c1[...], preferred_element_type=f32) + c1b[...], 0.0)
        v_ref[...] = jnp.dot(hc, c2[...], preferred_element_type=f32) + c2b[...]

    return pl.pallas_call(
        body,
        out_shape=[jax.ShapeDtypeStruct((GRAPHS, ACT), f32),
                   jax.ShapeDtypeStruct((GRAPHS, 1), f32)])(
            emb, den, aw1t, ab1, aw2t, ab2, cw1t, cb1, cw2t, cb2)


def _prep_edges(ei, epad, nsrc):
    e = ei.shape[1]
    src = ei[0].astype(i32)
    dst = ei[1].astype(i32)
    pad = epad + 2048 - e
    ar = jnp.arange(pad, dtype=i32)
    srcp = jnp.concatenate([src, ar % nsrc])
    dstp = jnp.concatenate([dst, NPHYS + (ar % 64)])
    return srcp, dstp.reshape(-1, 128)


def kernel(x_phys, x_log, x_gate, ei_intra, ei_inter, ei_mapped, ei_part,
           ei_dep, batch, params):
    del x_gate, ei_part, ei_dep

    si, di = _prep_edges(ei_intra, 819200, NPHYS)
    se, de = _prep_edges(ei_inter, 229376, NPHYS)
    sm, dm = _prep_edges(ei_mapped, 32768, NLOG)

    cnts = _counts_call(di, de, dm, 819200, 229376, 32768)
    recip = _tc_recip(cnts)

    xp_pad = jnp.pad(x_phys, ((0, NPHYS_PAD - NPHYS), (0, 3)))
    xl_pad = jnp.pad(x_log, ((0, NLOG_PAD - NLOG), (0, 7)))
    pe, le = params['phys_enc'], params['log_enc']
    wp8 = jnp.pad(pe['W'].T, ((0, 3), (0, 0)))
    wl8 = jnp.pad(le['W'].T, ((0, 7), (0, 0)))
    xp = _tc_encode(xp_pad, wp8, pe['b'].reshape(1, HID))
    xl = _tc_encode(xl_pad, wl8, le['b'].reshape(1, HID))
    xp2 = xp.reshape(2 * NPHYS_PAD, 32)
    xl2 = xl.reshape(2 * NLOG_PAD, 32)

    s1i = _segsum_call(xp2, si, di, 819200)
    s1e = _segsum_call(xp2, se, de, 229376)
    s1m = _segsum_call(xl2, sm, dm, 32768)
    c1 = params['conv1']
    wr1 = (c1['intra']['Wr'] + c1['inter']['Wr'] + c1['mapped']['Wr']).T
    b1 = (c1['intra']['bl'] + c1['inter']['bl'] + c1['mapped']['bl']).reshape(1, HID)
    p1 = _tc_combine(s1i[0], s1i[1], s1e[0], s1e[1], s1m[0], s1m[1], recip, xp,
                     c1['intra']['Wl'].T, c1['inter']['Wl'].T,
                     c1['mapped']['Wl'].T, wr1, b1)

    p12 = p1.reshape(2 * NPHYS, 32)
    s2i = _segsum_call(p12, si, di, 819200)
    s2e = _segsum_call(p12, se, de, 229376)
    c2 = params['conv2']
    wr2 = (c2['intra']['Wr'] + c2['inter']['Wr'] + c2['mapped']['Wr']).T
    b2 = (c2['intra']['bl'] + c2['inter']['bl'] + c2['mapped']['bl']).reshape(1, HID)
    p2 = _tc_combine(s2i[0], s2i[1], s2e[0], s2e[1], s1m[0], s1m[1], recip, p1,
                     c2['intra']['Wl'].T, c2['inter']['Wl'].T,
                     c2['mapped']['Wl'].T, wr2, b2)

    batch_col = batch.astype(i32).reshape(NPHYS, 1)
    pg = params['pool_gate']
    gsc, m = _tc_pool_a(p2, pg['W'].T, pg['b'].reshape(1, 1), batch_col)
    den, emb = _tc_pool_b(gsc, p2, batch_col, m)
    a, c = params['actor'], params['critic']
    logits, value = _tc_pool_c(
        emb, den, a['W1'].T, a['b1'].reshape(1, HID), a['W2'].T,
        a['b2'].reshape(1, -1), c['W1'].T, c['b1'].reshape(1, HID),
        c['W2'].T, c['b2'].reshape(1, 1))
    return logits, value
